# Initial kernel scaffold; baseline (speedup 1.0000x reference)
import jax
import jax.numpy as jnp
from jax.experimental import pallas as pl


def kernel(anchor, pos):
    raise NotImplementedError("write your pallas kernel here")


# speedup vs baseline: 2.1899x; 2.1899x over previous
import functools

import jax
import jax.numpy as jnp
from jax import lax
from jax.experimental import pallas as pl
from jax.experimental.pallas import tpu as pltpu


def _nt_kernel(x_row_ref, x_col_ref, out_ref, s_acc, p_acc, *,
               inv_temp, tm, tn, half_b):
    ri = pl.program_id(0)
    cj = pl.program_id(1)

    @pl.when(cj == 0)
    def _init():
        s_acc[...] = jnp.zeros_like(s_acc)
        p_acc[...] = jnp.zeros_like(p_acc)

    x_r = x_row_ref[...]
    x_c = x_col_ref[...]
    dn = (((1,), (1,)), ((), ()))
    s = lax.dot_general(x_r, x_c, dn,
                        preferred_element_type=jnp.float32)

    e = jnp.exp(s * inv_temp - inv_temp)
    s_acc[...] += e.sum(axis=1, keepdims=True)

    r0 = ri * tm
    c0 = cj * tn

    @pl.when((c0 < r0 + tm) & (c0 + tn > r0))
    def _diag():
        dmask = (lax.broadcasted_iota(jnp.int32, (tm, tn), 0) + r0 ==
                 lax.broadcasted_iota(jnp.int32, (tm, tn), 1) + c0)
        s_acc[...] -= jnp.where(dmask, e, 0.0).sum(axis=1, keepdims=True)

    off = jnp.where(r0 < half_b, half_b, -half_b)
    pc0 = r0 + off

    @pl.when((c0 < pc0 + tm) & (c0 + tn > pc0))
    def _pos():
        pmask = (lax.broadcasted_iota(jnp.int32, (tm, tn), 1) + c0 ==
                 lax.broadcasted_iota(jnp.int32, (tm, tn), 0) + r0 + off)
        p_acc[...] += (jnp.where(pmask, s, 0.0)
                       .sum(axis=1, keepdims=True)) * inv_temp

    @pl.when(cj == pl.num_programs(1) - 1)
    def _finalize():
        out_ref[...] = inv_temp + jnp.log(s_acc[...]) - p_acc[...]


def _ntxent(anchor, pos, temperature=0.1, tm=2048, tn=512):
    b, d = anchor.shape
    two_b = 2 * b
    tm = min(tm, two_b)
    tn = min(tn, two_b)
    assert two_b % tm == 0 and two_b % tn == 0 and b % tm == 0

    x = jnp.concatenate([anchor, pos], axis=0).astype(jnp.bfloat16)
    nb_r = two_b // tm
    nb_c = two_b // tn

    body = functools.partial(_nt_kernel, inv_temp=float(1.0 / temperature),
                             tm=tm, tn=tn, half_b=b)
    row_losses = pl.pallas_call(
        body,
        out_shape=jax.ShapeDtypeStruct((two_b, 1), jnp.float32),
        grid=(nb_r, nb_c),
        in_specs=[
            pl.BlockSpec((tm, d), lambda ri, cj: (ri, 0)),
            pl.BlockSpec((tn, d), lambda ri, cj: (cj, 0)),
        ],
        out_specs=pl.BlockSpec((tm, 1), lambda ri, cj: (ri, 0)),
        scratch_shapes=[
            pltpu.VMEM((tm, 1), jnp.float32),
            pltpu.VMEM((tm, 1), jnp.float32),
        ],
        compiler_params=pltpu.CompilerParams(
            dimension_semantics=("parallel", "arbitrary")),
    )(x, x)
    return jnp.mean(row_losses)


def kernel(anchor, pos):
    return _ntxent(anchor, pos, temperature=0.1)


# speedup vs baseline: 2.5326x; 1.1565x over previous
import functools

import jax
import jax.numpy as jnp
from jax import lax
from jax.experimental import pallas as pl
from jax.experimental.pallas import tpu as pltpu


_LOG2E = 1.4426950408889634


def _nt_kernel(x_row_ref, x_col_ref, out_ref, s_acc, p_acc, *,
               inv_temp, tm, tn, half_b):
    ri = pl.program_id(0)
    cj = pl.program_id(1)

    @pl.when(cj == 0)
    def _init():
        s_acc[...] = jnp.zeros_like(s_acc)
        p_acc[...] = jnp.zeros_like(p_acc)

    x_r = x_row_ref[...]
    x_c = x_col_ref[...]
    dn = (((1,), (1,)), ((), ()))
    s = lax.dot_general(x_r, x_c, dn,
                        preferred_element_type=jnp.float32)

    e = jnp.exp2(s - jnp.float32(inv_temp * _LOG2E))
    s_acc[...] += e.sum(axis=1, keepdims=True)

    r0 = ri * tm
    c0 = cj * tn

    @pl.when((c0 < r0 + tm) & (c0 + tn > r0))
    def _diag():
        dmask = (lax.broadcasted_iota(jnp.int32, (tm, tn), 0) + r0 ==
                 lax.broadcasted_iota(jnp.int32, (tm, tn), 1) + c0)
        s_acc[...] -= jnp.where(dmask, e, 0.0).sum(axis=1, keepdims=True)

    off = jnp.where(r0 < half_b, half_b, -half_b)
    pc0 = r0 + off

    @pl.when((c0 < pc0 + tm) & (c0 + tn > pc0))
    def _pos():
        pmask = (lax.broadcasted_iota(jnp.int32, (tm, tn), 1) + c0 ==
                 lax.broadcasted_iota(jnp.int32, (tm, tn), 0) + r0 + off)
        p_acc[...] += (jnp.where(pmask, s, 0.0)
                       .sum(axis=1, keepdims=True)) * jnp.float32(0.6931471805599453)

    @pl.when(cj == pl.num_programs(1) - 1)
    def _finalize():
        out_ref[...] = inv_temp + jnp.log(s_acc[...]) - p_acc[...]


def _ntxent(anchor, pos, temperature=0.1, tm=2048, tn=512):
    b, d = anchor.shape
    two_b = 2 * b
    tm = min(tm, two_b)
    tn = min(tn, two_b)
    assert two_b % tm == 0 and two_b % tn == 0 and b % tm == 0

    gamma = float((1.0 / temperature) * _LOG2E) ** 0.5
    x = (jnp.concatenate([anchor, pos], axis=0) * gamma).astype(jnp.bfloat16)
    nb_r = two_b // tm
    nb_c = two_b // tn

    body = functools.partial(_nt_kernel, inv_temp=float(1.0 / temperature),
                             tm=tm, tn=tn, half_b=b)
    row_losses = pl.pallas_call(
        body,
        out_shape=jax.ShapeDtypeStruct((two_b, 1), jnp.float32),
        grid=(nb_r, nb_c),
        in_specs=[
            pl.BlockSpec((tm, d), lambda ri, cj: (ri, 0)),
            pl.BlockSpec((tn, d), lambda ri, cj: (cj, 0)),
        ],
        out_specs=pl.BlockSpec((tm, 1), lambda ri, cj: (ri, 0)),
        scratch_shapes=[
            pltpu.VMEM((tm, 1), jnp.float32),
            pltpu.VMEM((tm, 1), jnp.float32),
        ],
        compiler_params=pltpu.CompilerParams(
            dimension_semantics=("parallel", "arbitrary")),
    )(x, x)
    return jnp.mean(row_losses)


def kernel(anchor, pos):
    return _ntxent(anchor, pos, temperature=0.1)


# speedup vs baseline: 2.9176x; 1.1520x over previous
import functools

import jax
import jax.numpy as jnp
from jax import lax
from jax.experimental import pallas as pl
from jax.experimental.pallas import tpu as pltpu


_LOG2E = 1.4426950408889634


def _nt_kernel(x_row_ref, x_col_ref, out_ref, s_acc, p_acc, *,
               inv_temp, tm, tn, half_b):
    ri = pl.program_id(0)
    cj = pl.program_id(1)

    @pl.when(cj == 0)
    def _init():
        s_acc[...] = jnp.zeros_like(s_acc)
        p_acc[...] = jnp.zeros_like(p_acc)

    x_r = x_row_ref[...]
    x_c = x_col_ref[...]
    dn = (((1,), (1,)), ((), ()))
    s = lax.dot_general(x_r, x_c, dn,
                        preferred_element_type=jnp.float32)

    e = jnp.exp2(s - jnp.float32(inv_temp * _LOG2E))
    s_acc[...] += e.sum(axis=1, keepdims=True)

    r0 = ri * tm
    c0 = cj * tn

    @pl.when((c0 < r0 + tm) & (c0 + tn > r0))
    def _diag():
        dmask = (lax.broadcasted_iota(jnp.int32, (tm, tn), 0) + r0 ==
                 lax.broadcasted_iota(jnp.int32, (tm, tn), 1) + c0)
        s_acc[...] -= jnp.where(dmask, e, 0.0).sum(axis=1, keepdims=True)

    off = jnp.where(r0 < half_b, half_b, -half_b)
    pc0 = r0 + off

    @pl.when((c0 < pc0 + tm) & (c0 + tn > pc0))
    def _pos():
        pmask = (lax.broadcasted_iota(jnp.int32, (tm, tn), 1) + c0 ==
                 lax.broadcasted_iota(jnp.int32, (tm, tn), 0) + r0 + off)
        p_acc[...] += (jnp.where(pmask, s, 0.0)
                       .sum(axis=1, keepdims=True)) * jnp.float32(0.6931471805599453)

    @pl.when(cj == pl.num_programs(1) - 1)
    def _finalize():
        out_ref[...] = inv_temp + jnp.log(s_acc[...]) - p_acc[...]


def _ntxent(anchor, pos, temperature=0.1, tm=2048, tn=512):
    b, d = anchor.shape
    two_b = 2 * b
    tm = min(tm, two_b)
    tn = min(tn, two_b)
    assert two_b % tm == 0 and two_b % tn == 0 and b % tm == 0

    gamma = float((1.0 / temperature) * _LOG2E) ** 0.5
    x = (jnp.concatenate([anchor, pos], axis=0) * gamma).astype(jnp.bfloat16)
    nb_r = two_b // tm
    nb_c = two_b // tn

    body = functools.partial(_nt_kernel, inv_temp=float(1.0 / temperature),
                             tm=tm, tn=tn, half_b=b)
    row_losses = pl.pallas_call(
        body,
        out_shape=jax.ShapeDtypeStruct((two_b, 1), jnp.float32),
        grid=(nb_r, nb_c),
        in_specs=[
            pl.BlockSpec((tm, d), lambda ri, cj: (ri, 0)),
            pl.BlockSpec((tn, d), lambda ri, cj: (cj, 0)),
        ],
        out_specs=pl.BlockSpec((tm, 1), lambda ri, cj: (ri, 0)),
        scratch_shapes=[
            pltpu.VMEM((tm, 1), jnp.float32),
            pltpu.VMEM((tm, 1), jnp.float32),
        ],
        compiler_params=pltpu.CompilerParams(
            dimension_semantics=("parallel", "arbitrary")),
    )(x, x)
    return jnp.mean(row_losses)


def kernel(anchor, pos):
    return _ntxent(anchor, pos, temperature=0.1, tm=2048, tn=1024)
